# baseline (speedup 1.0000x reference)
import functools

import jax
import jax.numpy as jnp
from jax import lax
from jax.experimental import pallas as pl
from jax.experimental.pallas import tpu as pltpu
from jax.experimental.pallas import tpu_sc as plsc

SZ = 3072
D = 1024
A = 31337 % SZ
B = 1000003 % SZ


def _matmul_body(tab_ref, w_ref, out_ref):
    out_ref[...] = lax.dot_general(
        tab_ref[...], w_ref[...],
        (((1,), (1,)), ((), ())),
        preferred_element_type=jnp.float32,
    )


def _project_table(tab, proj_w):
    m = tab.shape[0]
    bm = 512
    return pl.pallas_call(
        _matmul_body,
        grid=(m // bm,),
        in_specs=[
            pl.BlockSpec((bm, D), lambda i: (i, 0)),
            pl.BlockSpec((D, D), lambda i: (0, 0)),
        ],
        out_specs=pl.BlockSpec((bm, D), lambda i: (i, 0)),
        out_shape=jax.ShapeDtypeStruct((m, D), jnp.float32),
    )(tab, proj_w)


@functools.cache
def _make_gather(n):
    info = plsc.get_sparse_core_info()
    nc, ns, lanes = info.num_cores, info.num_subcores, info.num_lanes
    nw = nc * ns
    n_per_w = n // nw
    n_chunks = n_per_w // lanes

    mesh = plsc.VectorSubcoreMesh(core_axis_name="c", subcore_axis_name="s")

    nbuf = 3
    crows = 32
    n_chunks = n_per_w // crows

    @functools.partial(
        pl.kernel,
        mesh=mesh,
        out_type=jax.ShapeDtypeStruct((n, D), jnp.float32),
        scratch_types=[
            pltpu.VMEM((n_per_w,), jnp.int32),
            pltpu.VMEM((n_per_w,), jnp.int32),
            pltpu.VMEM((n_per_w,), jnp.int32),
        ]
        + [pltpu.VMEM((crows, D), jnp.float32) for _ in range(nbuf)]
        + [pltpu.SemaphoreType.DMA for _ in range(2 * nbuf)],
    )
    def gather_kernel(t_hbm, prev_hbm, tabp_hbm, out_hbm,
                      t_v, prev_v, idx_v, *bufsem):
        rows = bufsem[:nbuf]
        sg = bufsem[nbuf:2 * nbuf]
        so = bufsem[2 * nbuf:]
        wid = lax.axis_index("s") * nc + lax.axis_index("c")
        base = wid * n_per_w
        pltpu.sync_copy(t_hbm.at[pl.ds(base, n_per_w)], t_v)
        pltpu.sync_copy(prev_hbm.at[pl.ds(base, n_per_w)], prev_v)

        def hash_body(i, _):
            tv = t_v[pl.ds(i * lanes, lanes)]
            pv = prev_v[pl.ds(i * lanes, lanes)]
            idx_v[pl.ds(i * lanes, lanes)] = ((tv % SZ) * A + (pv % SZ) * B) % SZ
            return 0

        lax.fori_loop(0, n_per_w // lanes, hash_body, 0)

        def gather_dma(k, c):
            return pltpu.make_async_copy(
                tabp_hbm.at[idx_v.at[pl.ds(c * crows, crows)]], rows[k], sg[k])

        def out_dma(k, c):
            return pltpu.make_async_copy(
                rows[k], out_hbm.at[pl.ds(base + c * crows, crows)], so[k])

        def body(j, _):
            for k in range(nbuf):
                @pl.when(j > 0)
                def _drain(k=k):
                    out_dma(k, 0).wait()

                gather_dma(k, j * nbuf + k).start()

            for k in range(nbuf):
                gather_dma(k, 0).wait()
                out_dma(k, j * nbuf + k).start()
            return 0

        lax.fori_loop(0, n_chunks // nbuf, body, 0)
        for k in range(nbuf):
            out_dma(k, 0).wait()

    return gather_kernel


def kernel(t, tab, proj_w):
    bsz, seq = t.shape
    n = bsz * seq
    tabp = _project_table(tab, proj_w)
    tf = t.reshape(n)
    prevf = jnp.pad(t[:, :-1], ((0, 0), (1, 0))).reshape(n)
    outf = _make_gather(n)(tf, prevf, tabp)
    return outf.reshape(bsz, seq, D)


# speedup vs baseline: 1.0257x; 1.0257x over previous
import functools

import jax
import jax.numpy as jnp
from jax import lax
from jax.experimental import pallas as pl
from jax.experimental.pallas import tpu as pltpu
from jax.experimental.pallas import tpu_sc as plsc

SZ = 3072
D = 1024
A = 31337 % SZ
B = 1000003 % SZ


def _matmul_body(tab_ref, w_ref, out_ref):
    out_ref[...] = lax.dot_general(
        tab_ref[...], w_ref[...],
        (((1,), (1,)), ((), ())),
        preferred_element_type=jnp.float32,
    )


def _project_table(tab, proj_w):
    m = tab.shape[0]
    bm = 512
    return pl.pallas_call(
        _matmul_body,
        grid=(m // bm,),
        in_specs=[
            pl.BlockSpec((bm, D), lambda i: (i, 0)),
            pl.BlockSpec((D, D), lambda i: (0, 0)),
        ],
        out_specs=pl.BlockSpec((bm, D), lambda i: (i, 0)),
        out_shape=jax.ShapeDtypeStruct((m, D), jnp.float32),
    )(tab, proj_w)


@functools.cache
def _make_gather(n):
    info = plsc.get_sparse_core_info()
    nc, ns, lanes = info.num_cores, info.num_subcores, info.num_lanes
    nw = nc * ns
    n_per_w = n // nw
    n_chunks = n_per_w // lanes

    mesh = plsc.VectorSubcoreMesh(core_axis_name="c", subcore_axis_name="s")

    nbuf = 6
    crows = 16
    n_chunks = n_per_w // crows

    @functools.partial(
        pl.kernel,
        mesh=mesh,
        out_type=jax.ShapeDtypeStruct((n, D), jnp.float32),
        scratch_types=[
            pltpu.VMEM((n_per_w,), jnp.int32),
            pltpu.VMEM((n_per_w,), jnp.int32),
            pltpu.VMEM((n_per_w,), jnp.int32),
        ]
        + [pltpu.VMEM((crows, D), jnp.float32) for _ in range(nbuf)]
        + [pltpu.SemaphoreType.DMA for _ in range(2 * nbuf)],
    )
    def gather_kernel(t_hbm, prev_hbm, tabp_hbm, out_hbm,
                      t_v, prev_v, idx_v, *bufsem):
        rows = bufsem[:nbuf]
        sg = bufsem[nbuf:2 * nbuf]
        so = bufsem[2 * nbuf:]
        wid = lax.axis_index("s") * nc + lax.axis_index("c")
        base = wid * n_per_w
        pltpu.sync_copy(t_hbm.at[pl.ds(base, n_per_w)], t_v)
        pltpu.sync_copy(prev_hbm.at[pl.ds(base, n_per_w)], prev_v)

        def hash_body(i, _):
            tv = t_v[pl.ds(i * lanes, lanes)]
            pv = prev_v[pl.ds(i * lanes, lanes)]
            idx_v[pl.ds(i * lanes, lanes)] = ((tv % SZ) * A + (pv % SZ) * B) % SZ
            return 0

        lax.fori_loop(0, n_per_w // lanes, hash_body, 0)

        def gather_dma(k, c):
            return pltpu.make_async_copy(
                tabp_hbm.at[idx_v.at[pl.ds(c * crows, crows)]], rows[k], sg[k])

        def out_dma(k, c):
            return pltpu.make_async_copy(
                rows[k], out_hbm.at[pl.ds(base + c * crows, crows)], so[k])

        def body(j, _):
            for k in range(nbuf):
                @pl.when(j > 0)
                def _drain(k=k):
                    out_dma(k, 0).wait()

                gather_dma(k, j * nbuf + k).start()

            for k in range(nbuf):
                gather_dma(k, 0).wait()
                out_dma(k, j * nbuf + k).start()
            return 0

        lax.fori_loop(0, n_chunks // nbuf, body, 0)
        for k in range(nbuf):
            out_dma(k, 0).wait()

    return gather_kernel


def kernel(t, tab, proj_w):
    bsz, seq = t.shape
    n = bsz * seq
    tabp = _project_table(tab, proj_w)
    tf = t.reshape(n)
    prevf = jnp.pad(t[:, :-1], ((0, 0), (1, 0))).reshape(n)
    outf = _make_gather(n)(tf, prevf, tabp)
    return outf.reshape(bsz, seq, D)


# speedup vs baseline: 1.0404x; 1.0143x over previous
import functools

import jax
import jax.numpy as jnp
from jax import lax
from jax.experimental import pallas as pl
from jax.experimental.pallas import tpu as pltpu
from jax.experimental.pallas import tpu_sc as plsc

SZ = 3072
D = 1024
A = 31337 % SZ
B = 1000003 % SZ


def _matmul_body(tab_ref, w_ref, out_ref):
    out_ref[...] = lax.dot_general(
        tab_ref[...].astype(jnp.bfloat16), w_ref[...].astype(jnp.bfloat16),
        (((1,), (1,)), ((), ())),
        preferred_element_type=jnp.float32,
    )


def _project_table(tab, proj_w):
    m = tab.shape[0]
    bm = 512
    return pl.pallas_call(
        _matmul_body,
        grid=(m // bm,),
        in_specs=[
            pl.BlockSpec((bm, D), lambda i: (i, 0)),
            pl.BlockSpec((D, D), lambda i: (0, 0)),
        ],
        out_specs=pl.BlockSpec((bm, D), lambda i: (i, 0)),
        out_shape=jax.ShapeDtypeStruct((m, D), jnp.float32),
    )(tab, proj_w)


@functools.cache
def _make_gather(n):
    info = plsc.get_sparse_core_info()
    nc, ns, lanes = info.num_cores, info.num_subcores, info.num_lanes
    nw = nc * ns
    n_per_w = n // nw
    n_chunks = n_per_w // lanes

    mesh = plsc.VectorSubcoreMesh(core_axis_name="c", subcore_axis_name="s")

    nbuf = 4
    crows = lanes
    n_chunks = n_per_w // crows

    @functools.partial(
        pl.kernel,
        mesh=mesh,
        out_type=jax.ShapeDtypeStruct((n, D), jnp.float32),
        scratch_types=[
            pltpu.VMEM((n_per_w,), jnp.int32),
            pltpu.VMEM((n_per_w,), jnp.int32),
        ]
        + [pltpu.VMEM((crows, D), jnp.float32) for _ in range(nbuf)]
        + [pltpu.SemaphoreType.DMA for _ in range(2 * nbuf)],
    )
    def gather_kernel(t_hbm, prev_hbm, tabp_hbm, out_hbm, t_v, prev_v, *bufsem):
        rows = bufsem[:nbuf]
        sg = bufsem[nbuf:2 * nbuf]
        so = bufsem[2 * nbuf:]
        wid = lax.axis_index("s") * nc + lax.axis_index("c")
        base = wid * n_per_w
        pltpu.sync_copy(t_hbm.at[pl.ds(base, n_per_w)], t_v)
        pltpu.sync_copy(prev_hbm.at[pl.ds(base, n_per_w)], prev_v)

        def hash_idx(i):
            tv = t_v[pl.ds(i * lanes, lanes)]
            pv = prev_v[pl.ds(i * lanes, lanes)]
            return ((tv % SZ) * A + (pv % SZ) * B) % SZ

        def out_dma(k, c):
            return pltpu.make_async_copy(
                rows[k], out_hbm.at[pl.ds(base + c * crows, crows)], so[k])

        def body(j, _):
            gs = []
            for k in range(nbuf):
                @pl.when(j > 0)
                def _drain(k=k):
                    out_dma(k, 0).wait()

                gs.append(pltpu.async_copy(
                    tabp_hbm.at[hash_idx(j * nbuf + k)], rows[k], sg[k]))
            for k in range(nbuf):
                gs[k].wait()
                out_dma(k, j * nbuf + k).start()
            return 0

        lax.fori_loop(0, n_chunks // nbuf, body, 0)
        for k in range(nbuf):
            out_dma(k, 0).wait()

    return gather_kernel


def kernel(t, tab, proj_w):
    bsz, seq = t.shape
    n = bsz * seq
    tabp = _project_table(tab, proj_w)
    tf = t.reshape(n)
    prevf = jnp.pad(t[:, :-1], ((0, 0), (1, 0))).reshape(n)
    outf = _make_gather(n)(tf, prevf, tabp)
    return outf.reshape(bsz, seq, D)


# speedup vs baseline: 1.0742x; 1.0325x over previous
import functools

import jax
import jax.numpy as jnp
from jax import lax
from jax.experimental import pallas as pl
from jax.experimental.pallas import tpu as pltpu
from jax.experimental.pallas import tpu_sc as plsc

SZ = 3072
D = 1024
A = 31337 % SZ
B = 1000003 % SZ


def _matmul_body(tab_ref, w_ref, out_ref):
    out_ref[...] = lax.dot_general(
        tab_ref[...], w_ref[...],
        (((1,), (1,)), ((), ())),
        preferred_element_type=jnp.float32,
    )


def _project_table(tab, proj_w):
    m = tab.shape[0]
    bm = 512
    return pl.pallas_call(
        _matmul_body,
        grid=(m // bm,),
        in_specs=[
            pl.BlockSpec((bm, D), lambda i: (i, 0)),
            pl.BlockSpec((D, D), lambda i: (0, 0)),
        ],
        out_specs=pl.BlockSpec((bm, D), lambda i: (i, 0)),
        out_shape=jax.ShapeDtypeStruct((m, D), jnp.float32),
    )(tab, proj_w)


@functools.cache
def _make_gather(n):
    info = plsc.get_sparse_core_info()
    nc, ns, lanes = info.num_cores, info.num_subcores, info.num_lanes
    nw = nc * ns
    n_per_w = n // nw
    n_chunks = n_per_w // lanes

    mesh = plsc.VectorSubcoreMesh(core_axis_name="c", subcore_axis_name="s")

    nbuf = 6
    crows = lanes
    n_chunks = n_per_w // crows

    @functools.partial(
        pl.kernel,
        mesh=mesh,
        out_type=jax.ShapeDtypeStruct((n, D), jnp.float32),
        scratch_types=[
            pltpu.VMEM((n_per_w,), jnp.int32),
            pltpu.VMEM((n_per_w,), jnp.int32),
        ]
        + [pltpu.VMEM((crows, D), jnp.float32) for _ in range(nbuf)]
        + [pltpu.SemaphoreType.DMA for _ in range(2 * nbuf)],
    )
    def gather_kernel(t_hbm, prev_hbm, tabp_hbm, out_hbm, t_v, prev_v, *bufsem):
        rows = bufsem[:nbuf]
        sg = bufsem[nbuf:2 * nbuf]
        so = bufsem[2 * nbuf:]
        wid = lax.axis_index("s") * nc + lax.axis_index("c")
        base = wid * n_per_w
        pltpu.sync_copy(t_hbm.at[pl.ds(base, n_per_w)], t_v)
        pltpu.sync_copy(prev_hbm.at[pl.ds(base, n_per_w)], prev_v)

        def hash_idx(i):
            tv = t_v[pl.ds(i * lanes, lanes)]
            pv = prev_v[pl.ds(i * lanes, lanes)]
            return ((tv % SZ) * A + (pv % SZ) * B) % SZ

        def out_dma(k, c):
            return pltpu.make_async_copy(
                rows[k], out_hbm.at[pl.ds(base + c * crows, crows)], so[k])

        def body(j, _):
            gs = []
            for k in range(nbuf):
                @pl.when(j > 0)
                def _drain(k=k):
                    out_dma(k, 0).wait()

                gs.append(pltpu.async_copy(
                    tabp_hbm.at[hash_idx(j * nbuf + k)], rows[k], sg[k]))
            for k in range(nbuf):
                gs[k].wait()
                out_dma(k, j * nbuf + k).start()
            return 0

        lax.fori_loop(0, n_chunks // nbuf, body, 0)
        for k in range(nbuf):
            out_dma(k, 0).wait()

    return gather_kernel


def kernel(t, tab, proj_w):
    bsz, seq = t.shape
    n = bsz * seq
    tabp = _project_table(tab, proj_w)
    tf = t.reshape(n)
    prevf = jnp.pad(t[:, :-1], ((0, 0), (1, 0))).reshape(n)
    outf = _make_gather(n)(tf, prevf, tabp)
    return outf.reshape(bsz, seq, D)
